# baseline (speedup 1.0000x reference)
import functools

import jax
import jax.numpy as jnp
from jax import lax
from jax.experimental import pallas as pl
from jax.experimental.pallas import tpu as pltpu
from jax.experimental.pallas import tpu_sc as plsc

CLASS_NUM = 100000
PART_NUM = 8
FEA_DIM = 32
BATCH = 16384
LAMBDA_C = 1.0
ROW = PART_NUM * FEA_DIM

NC = 2
NS = 16
NW = NC * NS
BPW = BATCH // NW
CH = 64
NCHUNK = BPW // CH
LANES = 16
VPR = ROW // LANES

TC_ROWS = 38560
SC_ROWS = CLASS_NUM - TC_ROWS
SW = SC_ROWS // NW
CHS = 96
BLKS = CHS // LANES
NPAIRS = SW // (2 * CHS)

_sc_mesh = plsc.VectorSubcoreMesh(core_axis_name="c", subcore_axis_name="s")


@functools.partial(
    pl.kernel,
    out_type=(
        jax.ShapeDtypeStruct((NW, LANES), jnp.float32),
        jax.ShapeDtypeStruct((NW, LANES), jnp.float32),
        jax.ShapeDtypeStruct((NW, LANES), jnp.float32),
    ),
    mesh=_sc_mesh,
    compiler_params=pltpu.CompilerParams(needs_layout_passes=False),
    scratch_types=[
        pltpu.VMEM((BPW,), jnp.int32),
        pltpu.VMEM((CH, ROW), jnp.float32),
        pltpu.VMEM((CH, ROW), jnp.float32),
        pltpu.VMEM((CH, ROW), jnp.float32),
        pltpu.VMEM((CH, ROW), jnp.float32),
        pltpu.VMEM((CHS * ROW,), jnp.float32),
        pltpu.VMEM((CHS * ROW,), jnp.float32),
        pltpu.VMEM((LANES,), jnp.float32),
        pltpu.VMEM((LANES,), jnp.float32),
        pltpu.VMEM((LANES,), jnp.float32),
        pltpu.SemaphoreType.DMA,
        pltpu.SemaphoreType.DMA,
        pltpu.SemaphoreType.DMA,
        pltpu.SemaphoreType.DMA,
        pltpu.SemaphoreType.DMA,
        pltpu.SemaphoreType.DMA,
    ],
)
def _sc_main(x_hbm, lbl_hbm, centers_hbm, cflat_hbm, loss_out, s1_out, s2_out,
             idx_v, gb0, gb1, xb0, xb1, sb0, sb1,
             acc_v, s1_v, s2_v,
             sg0, sg1, sx0, sx1, ss0, ss1):
    wid = lax.axis_index("s") * NC + lax.axis_index("c")

    base = wid * BPW
    pltpu.sync_copy(lbl_hbm.at[pl.ds(base, BPW)], idx_v)

    gbufs = (gb0, gb1)
    xbufs = (xb0, xb1)
    gsems = (sg0, sg1)
    xsems = (sx0, sx1)

    def start(c):
        slot = c % 2
        cbase = c * CH
        g = pltpu.async_copy(
            centers_hbm.at[idx_v.at[pl.ds(cbase, CH)]], gbufs[slot], gsems[slot])
        x = pltpu.async_copy(
            x_hbm.at[pl.ds(base + cbase, CH)], xbufs[slot], xsems[slot])
        return g, x

    acc = jnp.zeros((LANES,), jnp.float32)
    pend = start(0)
    for c in range(NCHUNK):
        nxt = start(c + 1) if c + 1 < NCHUNK else None
        gcopy, xcopy = pend
        gcopy.wait()
        xcopy.wait()
        gb = gbufs[c % 2]
        xb = xbufs[c % 2]

        def row_body(r, a, gb=gb, xb=xb):
            for v in range(VPR):
                xv = xb[r, pl.ds(v * LANES, LANES)]
                gv = gb[r, pl.ds(v * LANES, LANES)]
                d = xv - gv
                a = a + d * d
            return a

        acc = lax.fori_loop(0, CH, row_body, acc)
        pend = nxt

    acc_v[...] = acc
    pltpu.sync_copy(acc_v, loss_out.at[wid])

    sbase = TC_ROWS + wid * SW
    sbufs = (sb0, sb1)
    ssems = (ss0, ss1)

    def s_start(chunk, slot):
        pltpu.async_copy(
            cflat_hbm.at[pl.ds((sbase + chunk * CHS) * ROW, CHS * ROW)],
            sbufs[slot], ssems[slot])

    def s_wait(slot):
        pltpu.make_async_copy(
            cflat_hbm.at[pl.ds(sbase * ROW, CHS * ROW)],
            sbufs[slot], ssems[slot]).wait()

    def compute_chunk(buf, carry):
        def blk(b, carry, buf=buf):
            s1a, s2a = carry
            rowbase = (b * LANES + jnp.arange(LANES, dtype=jnp.int32)) * ROW
            for p in range(PART_NUM):
                idx = rowbase + (p * FEA_DIM)
                gp = plsc.load_gather(buf, [idx])
                for _ in range(1, FEA_DIM):
                    idx = idx + 1
                    gp = gp + plsc.load_gather(buf, [idx])
                s1a = s1a + gp
                s2a = s2a + gp * gp
            return (s1a, s2a)

        return lax.fori_loop(0, BLKS, blk, carry)

    s_start(0, 0)
    carry = (jnp.zeros((LANES,), jnp.float32), jnp.zeros((LANES,), jnp.float32))

    def pair_body(c, carry):
        s_start(2 * c + 1, 1)
        s_wait(0)
        carry = compute_chunk(sb0, carry)

        @pl.when(c < NPAIRS - 1)
        def _():
            s_start(2 * c + 2, 0)

        s_wait(1)
        carry = compute_chunk(sb1, carry)
        return carry

    s1a, s2a = lax.fori_loop(0, NPAIRS, pair_body, carry)
    s1_v[...] = s1a
    s2_v[...] = s2a
    pltpu.sync_copy(s1_v, s1_out.at[wid])
    pltpu.sync_copy(s2_v, s2_out.at[wid])


BC = 7712
GRID = TC_ROWS // BC


def _stats_body(c_ref, g_ref, s1_ref, s2_ref):
    blk = c_ref[...]
    g = lax.dot_general(blk, g_ref[...], (((1,), (0,)), ((), ())),
                        preferred_element_type=jnp.float32)
    i = pl.program_id(0)
    s1_ref[i, 0] = jnp.sum(blk)
    s2_ref[i, 0] = jnp.sum(g * g)


_stats_call = pl.pallas_call(
    _stats_body,
    grid=(GRID,),
    in_specs=[
        pl.BlockSpec((BC, ROW), lambda i: (i, 0)),
        pl.BlockSpec((ROW, PART_NUM), lambda i: (0, 0)),
    ],
    out_specs=[
        pl.BlockSpec((GRID, 1), lambda i: (0, 0), memory_space=pltpu.SMEM),
        pl.BlockSpec((GRID, 1), lambda i: (0, 0), memory_space=pltpu.SMEM),
    ],
    out_shape=[
        jax.ShapeDtypeStruct((GRID, 1), jnp.float32),
        jax.ShapeDtypeStruct((GRID, 1), jnp.float32),
    ],
)


def kernel(x, label, centers):
    lbl = label.astype(jnp.int32)
    x2 = x.reshape(BATCH, ROW)
    c2 = centers.reshape(CLASS_NUM, ROW)

    loss_p, s1sc, s2sc = _sc_main(x2, lbl, c2, c2.reshape(-1))
    gmat = (jnp.arange(ROW, dtype=jnp.int32)[:, None] // FEA_DIM
            == jnp.arange(PART_NUM, dtype=jnp.int32)[None, :]
            ).astype(jnp.float32)
    s1p, s2p = _stats_call(c2[:TC_ROWS], gmat)

    n_all = CLASS_NUM * PART_NUM * FEA_DIM
    s1 = jnp.sum(s1p) + jnp.sum(s1sc)
    s2 = jnp.sum(s2p) + jnp.sum(s2sc)
    center_mean = s1 / n_all
    mean_m2 = s2 / (CLASS_NUM * PART_NUM * FEA_DIM * FEA_DIM)
    center_var = mean_m2 - center_mean * center_mean
    center_loss = LAMBDA_C * jnp.sum(loss_p) / (BATCH * PART_NUM * FEA_DIM)
    return (x, center_loss, center_mean, center_var)


# speedup vs baseline: 1.2663x; 1.2663x over previous
import functools

import jax
import jax.numpy as jnp
from jax import lax
from jax.experimental import pallas as pl
from jax.experimental.pallas import tpu as pltpu
from jax.experimental.pallas import tpu_sc as plsc

CLASS_NUM = 100000
PART_NUM = 8
FEA_DIM = 32
BATCH = 16384
LAMBDA_C = 1.0
ROW = PART_NUM * FEA_DIM

NC = 2
NS = 16
NW = NC * NS
BPW = BATCH // NW
CH = 64
NCHUNK = BPW // CH
LANES = 16
VPR = ROW // LANES

TC_ROWS = 38560
SC_ROWS = CLASS_NUM - TC_ROWS
SW = SC_ROWS // NW
CHS = 96
BLKS = CHS // LANES
NPAIRS = SW // (2 * CHS)

_sc_mesh = plsc.VectorSubcoreMesh(core_axis_name="c", subcore_axis_name="s")


@functools.partial(
    pl.kernel,
    out_type=(
        jax.ShapeDtypeStruct((NW, LANES), jnp.float32),
        jax.ShapeDtypeStruct((NW, LANES), jnp.float32),
        jax.ShapeDtypeStruct((NW, LANES), jnp.float32),
    ),
    mesh=_sc_mesh,
    compiler_params=pltpu.CompilerParams(needs_layout_passes=False),
    scratch_types=[
        pltpu.VMEM((BPW,), jnp.int32),
        pltpu.VMEM((CH, ROW), jnp.float32),
        pltpu.VMEM((CH, ROW), jnp.float32),
        pltpu.VMEM((CH, ROW), jnp.float32),
        pltpu.VMEM((CH, ROW), jnp.float32),
        pltpu.VMEM((CHS * ROW,), jnp.float32),
        pltpu.VMEM((CHS * ROW,), jnp.float32),
        pltpu.VMEM((LANES,), jnp.float32),
        pltpu.VMEM((LANES,), jnp.float32),
        pltpu.VMEM((LANES,), jnp.float32),
        pltpu.SemaphoreType.DMA,
        pltpu.SemaphoreType.DMA,
        pltpu.SemaphoreType.DMA,
        pltpu.SemaphoreType.DMA,
        pltpu.SemaphoreType.DMA,
        pltpu.SemaphoreType.DMA,
    ],
)
def _sc_main(x_hbm, lbl_hbm, centers_hbm, cflat_hbm, loss_out, s1_out, s2_out,
             idx_v, gb0, gb1, xb0, xb1, sb0, sb1,
             acc_v, s1_v, s2_v,
             sg0, sg1, sx0, sx1, ss0, ss1):
    wid = lax.axis_index("s") * NC + lax.axis_index("c")

    base = wid * BPW
    pltpu.sync_copy(lbl_hbm.at[pl.ds(base, BPW)], idx_v)

    gbufs = (gb0, gb1)
    xbufs = (xb0, xb1)
    gsems = (sg0, sg1)
    xsems = (sx0, sx1)

    def start(c):
        slot = c % 2
        cbase = c * CH
        g = pltpu.async_copy(
            centers_hbm.at[idx_v.at[pl.ds(cbase, CH)]], gbufs[slot], gsems[slot])
        x = pltpu.async_copy(
            x_hbm.at[pl.ds(base + cbase, CH)], xbufs[slot], xsems[slot])
        return g, x

    acc = jnp.zeros((LANES,), jnp.float32)
    pend = start(0)
    for c in range(NCHUNK):
        nxt = start(c + 1) if c + 1 < NCHUNK else None
        gcopy, xcopy = pend
        gcopy.wait()
        xcopy.wait()
        gb = gbufs[c % 2]
        xb = xbufs[c % 2]

        def row_body(r, a, gb=gb, xb=xb):
            for v in range(VPR):
                xv = xb[r, pl.ds(v * LANES, LANES)]
                gv = gb[r, pl.ds(v * LANES, LANES)]
                d = xv - gv
                a = a + d * d
            return a

        acc = lax.fori_loop(0, CH, row_body, acc)
        pend = nxt

    acc_v[...] = acc
    pltpu.sync_copy(acc_v, loss_out.at[wid])

    sbase = TC_ROWS + wid * SW
    sbufs = (sb0, sb1)
    ssems = (ss0, ss1)

    def s_start(chunk, slot):
        pltpu.async_copy(
            cflat_hbm.at[pl.ds((sbase + chunk * CHS) * ROW, CHS * ROW)],
            sbufs[slot], ssems[slot])

    def s_wait(slot):
        pltpu.make_async_copy(
            cflat_hbm.at[pl.ds(sbase * ROW, CHS * ROW)],
            sbufs[slot], ssems[slot]).wait()

    lane15 = jnp.arange(LANES, dtype=jnp.int32) == (LANES - 1)
    zerov = jnp.zeros((LANES,), jnp.float32)

    def compute_chunk(buf, carry):
        def rowb(r, carry, buf=buf):
            s1a, s2a = carry
            off = r * ROW
            for p in range(PART_NUM):
                a = buf[pl.ds(off + p * FEA_DIM, LANES)]
                b = buf[pl.ds(off + p * FEA_DIM + LANES, LANES)]
                pre = plsc.cumsum(a + b)
                s1a = s1a + jnp.where(lane15, pre, zerov)
                s2a = s2a + jnp.where(lane15, pre * pre, zerov)
            return (s1a, s2a)

        return lax.fori_loop(0, CHS, rowb, carry)

    s_start(0, 0)
    carry = (jnp.zeros((LANES,), jnp.float32), jnp.zeros((LANES,), jnp.float32))

    def pair_body(c, carry):
        s_start(2 * c + 1, 1)
        s_wait(0)
        carry = compute_chunk(sb0, carry)

        @pl.when(c < NPAIRS - 1)
        def _():
            s_start(2 * c + 2, 0)

        s_wait(1)
        carry = compute_chunk(sb1, carry)
        return carry

    s1a, s2a = lax.fori_loop(0, NPAIRS, pair_body, carry)
    s1_v[...] = s1a
    s2_v[...] = s2a
    pltpu.sync_copy(s1_v, s1_out.at[wid])
    pltpu.sync_copy(s2_v, s2_out.at[wid])


BC = 7712
GRID = TC_ROWS // BC


def _stats_body(c_ref, g_ref, s1_ref, s2_ref):
    blk = c_ref[...]
    g = lax.dot_general(blk, g_ref[...], (((1,), (0,)), ((), ())),
                        preferred_element_type=jnp.float32)
    i = pl.program_id(0)
    s1_ref[i, 0] = jnp.sum(blk)
    s2_ref[i, 0] = jnp.sum(g * g)


_stats_call = pl.pallas_call(
    _stats_body,
    grid=(GRID,),
    in_specs=[
        pl.BlockSpec((BC, ROW), lambda i: (i, 0)),
        pl.BlockSpec((ROW, PART_NUM), lambda i: (0, 0)),
    ],
    out_specs=[
        pl.BlockSpec((GRID, 1), lambda i: (0, 0), memory_space=pltpu.SMEM),
        pl.BlockSpec((GRID, 1), lambda i: (0, 0), memory_space=pltpu.SMEM),
    ],
    out_shape=[
        jax.ShapeDtypeStruct((GRID, 1), jnp.float32),
        jax.ShapeDtypeStruct((GRID, 1), jnp.float32),
    ],
)


def kernel(x, label, centers):
    lbl = label.astype(jnp.int32)
    x2 = x.reshape(BATCH, ROW)
    c2 = centers.reshape(CLASS_NUM, ROW)

    loss_p, s1sc, s2sc = _sc_main(x2, lbl, c2, c2.reshape(-1))
    gmat = (jnp.arange(ROW, dtype=jnp.int32)[:, None] // FEA_DIM
            == jnp.arange(PART_NUM, dtype=jnp.int32)[None, :]
            ).astype(jnp.float32)
    s1p, s2p = _stats_call(c2[:TC_ROWS], gmat)

    n_all = CLASS_NUM * PART_NUM * FEA_DIM
    s1 = jnp.sum(s1p) + jnp.sum(s1sc)
    s2 = jnp.sum(s2p) + jnp.sum(s2sc)
    center_mean = s1 / n_all
    mean_m2 = s2 / (CLASS_NUM * PART_NUM * FEA_DIM * FEA_DIM)
    center_var = mean_m2 - center_mean * center_mean
    center_loss = LAMBDA_C * jnp.sum(loss_p) / (BATCH * PART_NUM * FEA_DIM)
    return (x, center_loss, center_mean, center_var)


# speedup vs baseline: 3.0638x; 2.4195x over previous
import functools

import jax
import jax.numpy as jnp
from jax import lax
from jax.experimental import pallas as pl
from jax.experimental.pallas import tpu as pltpu
from jax.experimental.pallas import tpu_sc as plsc

CLASS_NUM = 100000
PART_NUM = 8
FEA_DIM = 32
BATCH = 16384
LAMBDA_C = 1.0
ROW = PART_NUM * FEA_DIM

NC = 2
NS = 16
NW = NC * NS
BPW = BATCH // NW
CH = 64
NCHUNK = BPW // CH
LANES = 16
VPR = ROW // LANES

TC_ROWS = 44704
SC_ROWS = CLASS_NUM - TC_ROWS
SW = SC_ROWS // NW
CHS = 96
BLKS = CHS // LANES
NPAIRS = SW // (2 * CHS)

_sc_mesh = plsc.VectorSubcoreMesh(core_axis_name="c", subcore_axis_name="s")


@functools.partial(
    pl.kernel,
    out_type=(
        jax.ShapeDtypeStruct((NW, LANES), jnp.float32),
        jax.ShapeDtypeStruct((NW, LANES), jnp.float32),
        jax.ShapeDtypeStruct((NW, LANES), jnp.float32),
    ),
    mesh=_sc_mesh,
    compiler_params=pltpu.CompilerParams(needs_layout_passes=False),
    scratch_types=[
        pltpu.VMEM((BPW,), jnp.int32),
        pltpu.VMEM((CH, ROW), jnp.float32),
        pltpu.VMEM((CH, ROW), jnp.float32),
        pltpu.VMEM((CH, ROW), jnp.float32),
        pltpu.VMEM((CH, ROW), jnp.float32),
        pltpu.VMEM((CHS, ROW), jnp.float32),
        pltpu.VMEM((CHS, ROW), jnp.float32),
        pltpu.VMEM((LANES,), jnp.float32),
        pltpu.VMEM((LANES,), jnp.float32),
        pltpu.VMEM((LANES,), jnp.float32),
        pltpu.SemaphoreType.DMA,
        pltpu.SemaphoreType.DMA,
        pltpu.SemaphoreType.DMA,
        pltpu.SemaphoreType.DMA,
        pltpu.SemaphoreType.DMA,
        pltpu.SemaphoreType.DMA,
    ],
)
def _sc_main(x_hbm, lbl_hbm, centers_hbm, loss_out, s1_out, s2_out,
             idx_v, gb0, gb1, xb0, xb1, sb0, sb1,
             acc_v, s1_v, s2_v,
             sg0, sg1, sx0, sx1, ss0, ss1):
    wid = lax.axis_index("s") * NC + lax.axis_index("c")

    base = wid * BPW
    pltpu.sync_copy(lbl_hbm.at[pl.ds(base, BPW)], idx_v)

    gbufs = (gb0, gb1)
    xbufs = (xb0, xb1)
    gsems = (sg0, sg1)
    xsems = (sx0, sx1)

    def start(c):
        slot = c % 2
        cbase = c * CH
        g = pltpu.async_copy(
            centers_hbm.at[idx_v.at[pl.ds(cbase, CH)]], gbufs[slot], gsems[slot])
        x = pltpu.async_copy(
            x_hbm.at[pl.ds(base + cbase, CH)], xbufs[slot], xsems[slot])
        return g, x

    acc = jnp.zeros((LANES,), jnp.float32)
    pend = start(0)
    for c in range(NCHUNK):
        nxt = start(c + 1) if c + 1 < NCHUNK else None
        gcopy, xcopy = pend
        gcopy.wait()
        xcopy.wait()
        gb = gbufs[c % 2]
        xb = xbufs[c % 2]

        def row_body(r, a, gb=gb, xb=xb):
            for v in range(VPR):
                xv = xb[r, pl.ds(v * LANES, LANES)]
                gv = gb[r, pl.ds(v * LANES, LANES)]
                d = xv - gv
                a = a + d * d
            return a

        acc = lax.fori_loop(0, CH, row_body, acc)
        pend = nxt

    acc_v[...] = acc
    pltpu.sync_copy(acc_v, loss_out.at[wid])

    sbase = TC_ROWS + wid * SW
    sbufs = (sb0, sb1)
    ssems = (ss0, ss1)

    def s_start(chunk, slot):
        pltpu.async_copy(
            centers_hbm.at[pl.ds(sbase + chunk * CHS, CHS)],
            sbufs[slot], ssems[slot])

    def s_wait(slot):
        pltpu.make_async_copy(
            centers_hbm.at[pl.ds(sbase, CHS)],
            sbufs[slot], ssems[slot]).wait()

    lane15 = jnp.arange(LANES, dtype=jnp.int32) == (LANES - 1)
    zerov = jnp.zeros((LANES,), jnp.float32)

    def compute_chunk(buf, carry):
        def rowb(r, carry, buf=buf):
            s1a, s2a = carry
            for p in range(PART_NUM):
                a = buf[r, pl.ds(p * FEA_DIM, LANES)]
                b = buf[r, pl.ds(p * FEA_DIM + LANES, LANES)]
                pre = plsc.cumsum(a + b)
                s1a = s1a + jnp.where(lane15, pre, zerov)
                s2a = s2a + jnp.where(lane15, pre * pre, zerov)
            return (s1a, s2a)

        return lax.fori_loop(0, CHS, rowb, carry)

    s_start(0, 0)
    carry = (jnp.zeros((LANES,), jnp.float32), jnp.zeros((LANES,), jnp.float32))

    def pair_body(c, carry):
        s_start(2 * c + 1, 1)
        s_wait(0)
        carry = compute_chunk(sb0, carry)

        @pl.when(c < NPAIRS - 1)
        def _():
            s_start(2 * c + 2, 0)

        s_wait(1)
        carry = compute_chunk(sb1, carry)
        return carry

    s1a, s2a = lax.fori_loop(0, NPAIRS, pair_body, carry)
    s1_v[...] = s1a
    s2_v[...] = s2a
    pltpu.sync_copy(s1_v, s1_out.at[wid])
    pltpu.sync_copy(s2_v, s2_out.at[wid])


BC = 11176
GRID = TC_ROWS // BC


def _stats_body(c_ref, g_ref, s1_ref, s2_ref):
    blk = c_ref[...]
    g = lax.dot_general(blk, g_ref[...], (((1,), (0,)), ((), ())),
                        preferred_element_type=jnp.float32)
    i = pl.program_id(0)
    s1_ref[i, 0] = jnp.sum(blk)
    s2_ref[i, 0] = jnp.sum(g * g)


_stats_call = pl.pallas_call(
    _stats_body,
    grid=(GRID,),
    in_specs=[
        pl.BlockSpec((BC, ROW), lambda i: (i, 0)),
        pl.BlockSpec((ROW, PART_NUM), lambda i: (0, 0)),
    ],
    out_specs=[
        pl.BlockSpec((GRID, 1), lambda i: (0, 0), memory_space=pltpu.SMEM),
        pl.BlockSpec((GRID, 1), lambda i: (0, 0), memory_space=pltpu.SMEM),
    ],
    out_shape=[
        jax.ShapeDtypeStruct((GRID, 1), jnp.float32),
        jax.ShapeDtypeStruct((GRID, 1), jnp.float32),
    ],
)


def kernel(x, label, centers):
    lbl = label.astype(jnp.int32)
    x2 = x.reshape(BATCH, ROW)
    c2 = centers.reshape(CLASS_NUM, ROW)

    loss_p, s1sc, s2sc = _sc_main(x2, lbl, c2)
    gmat = (jnp.arange(ROW, dtype=jnp.int32)[:, None] // FEA_DIM
            == jnp.arange(PART_NUM, dtype=jnp.int32)[None, :]
            ).astype(jnp.float32)
    s1p, s2p = _stats_call(c2[:TC_ROWS], gmat)

    n_all = CLASS_NUM * PART_NUM * FEA_DIM
    s1 = jnp.sum(s1p) + jnp.sum(s1sc)
    s2 = jnp.sum(s2p) + jnp.sum(s2sc)
    center_mean = s1 / n_all
    mean_m2 = s2 / (CLASS_NUM * PART_NUM * FEA_DIM * FEA_DIM)
    center_var = mean_m2 - center_mean * center_mean
    center_loss = LAMBDA_C * jnp.sum(loss_p) / (BATCH * PART_NUM * FEA_DIM)
    return (x, center_loss, center_mean, center_var)


# speedup vs baseline: 4.0527x; 1.3228x over previous
import functools

import jax
import jax.numpy as jnp
from jax import lax
from jax.experimental import pallas as pl
from jax.experimental.pallas import tpu as pltpu
from jax.experimental.pallas import tpu_sc as plsc

CLASS_NUM = 100000
PART_NUM = 8
FEA_DIM = 32
BATCH = 16384
LAMBDA_C = 1.0
ROW = PART_NUM * FEA_DIM

NC = 2
NS = 16
NW = NC * NS
BPW = BATCH // NW
CH = 64
NCHUNK = BPW // CH
LANES = 16
VPR = ROW // LANES

_sc_mesh = plsc.VectorSubcoreMesh(core_axis_name="c", subcore_axis_name="s")


@functools.partial(
    pl.kernel,
    out_type=jax.ShapeDtypeStruct((NW, LANES), jnp.float32),
    mesh=_sc_mesh,
    compiler_params=pltpu.CompilerParams(needs_layout_passes=False),
    scratch_types=[
        pltpu.VMEM((BPW,), jnp.int32),
        pltpu.VMEM((CH, ROW), jnp.float32),
        pltpu.VMEM((CH, ROW), jnp.float32),
        pltpu.VMEM((CH, ROW), jnp.float32),
        pltpu.VMEM((CH, ROW), jnp.float32),
        pltpu.VMEM((LANES,), jnp.float32),
        pltpu.SemaphoreType.DMA,
        pltpu.SemaphoreType.DMA,
        pltpu.SemaphoreType.DMA,
        pltpu.SemaphoreType.DMA,
    ],
)
def _sc_loss(x_hbm, lbl_hbm, centers_hbm, loss_out,
             idx_v, gb0, gb1, xb0, xb1, acc_v,
             sg0, sg1, sx0, sx1):
    wid = lax.axis_index("s") * NC + lax.axis_index("c")
    base = wid * BPW
    pltpu.sync_copy(lbl_hbm.at[pl.ds(base, BPW)], idx_v)

    gbufs = (gb0, gb1)
    xbufs = (xb0, xb1)
    gsems = (sg0, sg1)
    xsems = (sx0, sx1)

    def start(c):
        slot = c % 2
        cbase = c * CH
        g = pltpu.async_copy(
            centers_hbm.at[idx_v.at[pl.ds(cbase, CH)]], gbufs[slot], gsems[slot])
        x = pltpu.async_copy(
            x_hbm.at[pl.ds(base + cbase, CH)], xbufs[slot], xsems[slot])
        return g, x

    acc = jnp.zeros((LANES,), jnp.float32)
    pend = start(0)
    for c in range(NCHUNK):
        nxt = start(c + 1) if c + 1 < NCHUNK else None
        gcopy, xcopy = pend
        gcopy.wait()
        xcopy.wait()
        gb = gbufs[c % 2]
        xb = xbufs[c % 2]

        def row_body(r, a, gb=gb, xb=xb):
            for v in range(VPR):
                xv = xb[r, pl.ds(v * LANES, LANES)]
                gv = gb[r, pl.ds(v * LANES, LANES)]
                d = xv - gv
                a = a + d * d
            return a

        acc = lax.fori_loop(0, CH, row_body, acc)
        pend = nxt

    acc_v[...] = acc
    pltpu.sync_copy(acc_v, loss_out.at[wid])


def _stats_body(c_ref, s1_ref, s2_ref):
    blk = c_ref[...]
    g = jnp.sum(blk, axis=1)
    i = pl.program_id(0)
    s1_ref[i, 0] = jnp.sum(blk)
    s2_ref[i, 0] = jnp.sum(g * g)


_stats_call = pl.pallas_call(
    _stats_body,
    grid=(PART_NUM,),
    in_specs=[pl.BlockSpec((1, FEA_DIM, CLASS_NUM), lambda i: (i, 0, 0))],
    out_specs=[
        pl.BlockSpec((PART_NUM, 1), lambda i: (0, 0), memory_space=pltpu.SMEM),
        pl.BlockSpec((PART_NUM, 1), lambda i: (0, 0), memory_space=pltpu.SMEM),
    ],
    out_shape=[
        jax.ShapeDtypeStruct((PART_NUM, 1), jnp.float32),
        jax.ShapeDtypeStruct((PART_NUM, 1), jnp.float32),
    ],
)


def kernel(x, label, centers):
    lbl = label.astype(jnp.int32)
    x2 = x.reshape(BATCH, ROW)
    c2 = centers.reshape(CLASS_NUM, ROW)
    ct = jnp.transpose(centers, (1, 2, 0))

    loss_p = _sc_loss(x2, lbl, c2)
    s1p, s2p = _stats_call(ct)

    n_all = CLASS_NUM * PART_NUM * FEA_DIM
    s1 = jnp.sum(s1p)
    s2 = jnp.sum(s2p)
    center_mean = s1 / n_all
    mean_m2 = s2 / (CLASS_NUM * PART_NUM * FEA_DIM * FEA_DIM)
    center_var = mean_m2 - center_mean * center_mean
    center_loss = LAMBDA_C * jnp.sum(loss_p) / (BATCH * PART_NUM * FEA_DIM)
    return (x, center_loss, center_mean, center_var)
